# baseline (speedup 1.0000x reference)
import jax
import jax.numpy as jnp
import numpy as np
from jax import lax
from jax.experimental import pallas as pl
from jax.experimental.pallas import tpu as pltpu
from jax.experimental.pallas import tpu_sc as plsc

_FEATURE_DIMS = [100000] * 26
_F = 26
_D = 32
_B = 4096
_NC, _NS = 2, 16
_NW = _NC * _NS
_BPW = _B // _NW
_RPW = _BPW * _F
_CW = 104
_EPS = 8
_SROWS = _EPS * _F
_NSTAGE = _BPW // _EPS

_OFFSETS_NP = np.concatenate([[0], np.cumsum(_FEATURE_DIMS)[:-1]]).astype(np.int32)

_P = np.arange(_RPW)
_OFFPAT_NP = _OFFSETS_NP[_P % _F]
_LT_NP = ((_P % _F) * _BPW + _P // _F).astype(np.int32)


def _body(x_hbm, off_hbm, lt_hbm, emb_hbm, lin_hbm, bias_hbm, out_hbm,
          idx_v, off_v, lt_v, lidx_v, buf_a, buf_b, lin_v, t_v, out_v,
          bias_v, sem_in, sem_a, sem_b, sem_l):
    c = lax.axis_index("c")
    s = lax.axis_index("s")
    wid = s * _NC + c

    d0 = pltpu.async_copy(x_hbm.at[pl.ds(wid * _RPW, _RPW)], idx_v, sem_in)
    d1 = pltpu.async_copy(off_hbm, off_v, sem_in)
    d2 = pltpu.async_copy(lt_hbm, lt_v, sem_in)
    d3 = pltpu.async_copy(bias_hbm, bias_v, sem_in)
    d0.wait()
    d1.wait()
    d2.wait()
    d3.wait()

    def idx_body(i, carry):
        sl = pl.ds(i * 16, 16)
        v = idx_v[sl] + off_v[sl]
        idx_v[sl] = v
        plsc.store_scatter(lidx_v, [lt_v[sl]], v)
        return carry

    lax.fori_loop(0, _RPW // 16, idx_body, 0)

    def fire_stage(st, buf, sem):
        return [
            pltpu.async_copy(emb_hbm.at[idx_v.at[pl.ds((2 * st + k) * _CW, _CW)]],
                             buf.at[pl.ds(k * _CW, _CW)], sem)
            for k in range(2)
        ]

    descs = [None] * _NSTAGE
    descs[0] = fire_stage(0, buf_a, sem_a)
    lin_descs = [
        pltpu.async_copy(lin_hbm.at[lidx_v.at[pl.ds(f * _BPW, _BPW)]],
                         lin_v.at[pl.ds(f * _BPW, _BPW)], sem_l)
        for f in range(_F)
    ]
    descs[1] = fire_stage(1, buf_b, sem_b)

    iota = lax.iota(jnp.int32, 16)
    zeros_f = jnp.zeros((16,), jnp.float32)
    bias_vec = bias_v[:]

    def compute_stage(buf, st):
        half = st % 2

        def elem_body(e, carry):
            s0 = s1 = ss0 = ss1 = zeros_f
            for f in range(_F):
                j = e * _F + f
                v0 = buf[j, pl.ds(0, 16)]
                v1 = buf[j, pl.ds(16, 16)]
                s0 = s0 + v0
                ss0 = ss0 + v0 * v0
                s1 = s1 + v1
                ss1 = ss1 + v1 * v1
            t = s0 * s0 + s1 * s1 - ss0 - ss1
            t_v[pl.ds((half * 8 + e) * 16, 16)] = t
            return carry

        lax.fori_loop(0, _EPS, elem_body, 0)

        if half == 1:
            grp = st // 2
            fm = zeros_f
            for l in range(16):
                fm = fm + plsc.load_gather(t_v, [iota * 16 + l])
            lin_acc = zeros_f
            for f in range(_F):
                lin_acc = lin_acc + lin_v[pl.ds(f * _BPW + grp * 16, 16)]
            r = lin_acc + bias_vec + 0.5 * fm
            out_v[pl.ds(grp * 16, 16)] = 1.0 / (1.0 + jnp.exp(-r))

    for st in range(_NSTAGE):
        buf = buf_a if st % 2 == 0 else buf_b
        sem = sem_a if st % 2 == 0 else sem_b
        for d in descs[st]:
            d.wait()
        if st == 1:
            for d in lin_descs:
                d.wait()
        compute_stage(buf, st)
        if st + 2 < _NSTAGE:
            descs[st + 2] = fire_stage(st + 2, buf, sem)

    pltpu.sync_copy(out_v, out_hbm.at[pl.ds(wid * _BPW, _BPW)])


@jax.jit
def kernel(x, emb_table, lin_table, bias):
    mesh = plsc.VectorSubcoreMesh(core_axis_name="c", subcore_axis_name="s",
                                  num_cores=_NC, num_subcores=_NS)
    kfn = pl.kernel(
        _body,
        out_type=jax.ShapeDtypeStruct((_B,), jnp.float32),
        mesh=mesh,
        compiler_params=pltpu.CompilerParams(
            needs_layout_passes=False,
            use_tc_tiling_on_sc=False,
        ),
        scratch_types=[
            pltpu.VMEM((_RPW,), jnp.int32),
            pltpu.VMEM((_RPW,), jnp.int32),
            pltpu.VMEM((_RPW,), jnp.int32),
            pltpu.VMEM((_RPW,), jnp.int32),
            pltpu.VMEM((_SROWS, _D), jnp.float32),
            pltpu.VMEM((_SROWS, _D), jnp.float32),
            pltpu.VMEM((_RPW,), jnp.float32),
            pltpu.VMEM((256,), jnp.float32),
            pltpu.VMEM((_BPW,), jnp.float32),
            pltpu.VMEM((16,), jnp.float32),
            pltpu.SemaphoreType.DMA,
            pltpu.SemaphoreType.DMA,
            pltpu.SemaphoreType.DMA,
            pltpu.SemaphoreType.DMA,
        ],
    )
    return kfn(x.reshape(-1), jnp.asarray(_OFFPAT_NP), jnp.asarray(_LT_NP),
               emb_table, lin_table.reshape(-1), jnp.broadcast_to(bias, (16,)))


# speedup vs baseline: 1.4451x; 1.4451x over previous
import jax
import jax.numpy as jnp
import numpy as np
from jax import lax
from jax.experimental import pallas as pl
from jax.experimental.pallas import tpu as pltpu
from jax.experimental.pallas import tpu_sc as plsc

_FEATURE_DIMS = [100000] * 26
_F = 26
_D = 32
_B = 4096
_NC, _NS = 2, 16
_NW = _NC * _NS
_BPW = _B // _NW
_RPW = _BPW * _F
_EPS = 8
_SROWS = _EPS * _F
_NSTAGE = _BPW // _EPS

_OFFSETS_NP = np.concatenate([[0], np.cumsum(_FEATURE_DIMS)[:-1]]).astype(np.int32)

_P = np.arange(_RPW)
_OFFPAT_NP = _OFFSETS_NP[_P % _F]
_LT_NP = ((_P % _F) * _BPW + _P // _F).astype(np.int32)


def _body_a(x_hbm, off_hbm, lt_hbm, lin_hbm, idx_out, lsum_out,
            idx_v, off_v, lt_v, lidx_v, lin_v, lsum_v, sem_in, sem_l):
    c = lax.axis_index("c")
    s = lax.axis_index("s")
    wid = s * _NC + c

    d0 = pltpu.async_copy(x_hbm.at[pl.ds(wid * _RPW, _RPW)], idx_v, sem_in)
    d1 = pltpu.async_copy(off_hbm, off_v, sem_in)
    d2 = pltpu.async_copy(lt_hbm, lt_v, sem_in)
    d0.wait()
    d1.wait()
    d2.wait()

    def idx_body(i, carry):
        sl = pl.ds(i * 16, 16)
        v = idx_v[sl] + off_v[sl]
        idx_v[sl] = v
        plsc.store_scatter(lidx_v, [lt_v[sl]], v)
        return carry

    lax.fori_loop(0, _RPW // 16, idx_body, 0)

    out_d = pltpu.async_copy(idx_v, idx_out.at[pl.ds(wid * _RPW, _RPW)], sem_in)

    lin_descs = [
        pltpu.async_copy(lin_hbm.at[lidx_v.at[pl.ds(f * _BPW, _BPW)]],
                         lin_v.at[pl.ds(f * _BPW, _BPW)], sem_l)
        for f in range(_F)
    ]
    for d in lin_descs:
        d.wait()

    for g in range(_BPW // 16):
        acc = jnp.zeros((16,), jnp.float32)
        for f in range(_F):
            acc = acc + lin_v[pl.ds(f * _BPW + g * 16, 16)]
        lsum_v[pl.ds(g * 16, 16)] = acc

    out_d.wait()
    pltpu.sync_copy(lsum_v, lsum_out.at[pl.ds(wid * _BPW, _BPW)])


def _body_b(emb_hbm, idx_hbm, lsum_hbm, bias_hbm, out_hbm,
            idx_v, buf, lins_v, t_v, out_v, bias_v, sem_in, sem):
    c = lax.axis_index("c")
    s = lax.axis_index("s")
    wid = s * _NC + c

    d0 = pltpu.async_copy(idx_hbm.at[pl.ds(wid * _RPW, _RPW)], idx_v, sem_in)
    d1 = pltpu.async_copy(lsum_hbm.at[pl.ds(wid * _BPW, _BPW)], lins_v, sem_in)
    d2 = pltpu.async_copy(bias_hbm, bias_v, sem_in)
    d0.wait()
    d1.wait()
    d2.wait()

    iota = lax.iota(jnp.int32, 16)
    zeros_f = jnp.zeros((16,), jnp.float32)
    bias_vec = bias_v[:]

    def issue_stage(st):
        par = st % 2
        halfoff = par * _SROWS

        def vec_body(i, carry):
            v = idx_v[pl.ds(st * _SROWS + i * 16, 16)]
            for l in range(16):
                r = v[l]
                pltpu.async_copy(
                    emb_hbm.at[pl.ds(r, 1), :],
                    buf.at[pl.ds(halfoff + i * 16 + l, 1), :],
                    sem.at[par],
                )
            return carry

        lax.fori_loop(0, _SROWS // 16, vec_body, 0)

    def drain_stage(st):
        par = st % 2
        halfoff = par * _SROWS
        pltpu.make_async_copy(
            emb_hbm.at[pl.ds(0, _SROWS), :],
            buf.at[pl.ds(halfoff, _SROWS), :],
            sem.at[par],
        ).wait()

    issue_stage(0)
    issue_stage(1)

    for st in range(_NSTAGE):
        half = st % 2
        halfoff = half * _SROWS
        drain_stage(st)

        def elem_body(e, carry):
            s0 = s1 = ss0 = ss1 = zeros_f
            for f in range(_F):
                j = halfoff + e * _F + f
                v0 = buf[j, pl.ds(0, 16)]
                v1 = buf[j, pl.ds(16, 16)]
                s0 = s0 + v0
                ss0 = ss0 + v0 * v0
                s1 = s1 + v1
                ss1 = ss1 + v1 * v1
            t = s0 * s0 + s1 * s1 - ss0 - ss1
            t_v[pl.ds((half * 8 + e) * 16, 16)] = t
            return carry

        lax.fori_loop(0, _EPS, elem_body, 0)

        if half == 1:
            grp = st // 2
            fm = zeros_f
            for l in range(16):
                fm = fm + plsc.load_gather(t_v, [iota * 16 + l])
            r = lins_v[pl.ds(grp * 16, 16)] + bias_vec + 0.5 * fm
            out_v[pl.ds(grp * 16, 16)] = 1.0 / (1.0 + jnp.exp(-r))

        if st + 2 < _NSTAGE:
            issue_stage(st + 2)

    pltpu.sync_copy(out_v, out_hbm.at[pl.ds(wid * _BPW, _BPW)])


@jax.jit
def kernel(x, emb_table, lin_table, bias):
    mesh = plsc.VectorSubcoreMesh(core_axis_name="c", subcore_axis_name="s",
                                  num_cores=_NC, num_subcores=_NS)
    kfn_a = pl.kernel(
        _body_a,
        out_type=(
            jax.ShapeDtypeStruct((_B * _F,), jnp.int32),
            jax.ShapeDtypeStruct((_B,), jnp.float32),
        ),
        mesh=mesh,
        compiler_params=pltpu.CompilerParams(
            needs_layout_passes=False,
            use_tc_tiling_on_sc=False,
        ),
        scratch_types=[
            pltpu.VMEM((_RPW,), jnp.int32),
            pltpu.VMEM((_RPW,), jnp.int32),
            pltpu.VMEM((_RPW,), jnp.int32),
            pltpu.VMEM((_RPW,), jnp.int32),
            pltpu.VMEM((_RPW,), jnp.float32),
            pltpu.VMEM((_BPW,), jnp.float32),
            pltpu.SemaphoreType.DMA,
            pltpu.SemaphoreType.DMA,
        ],
    )
    idx_all, lsum = kfn_a(x.reshape(-1), jnp.asarray(_OFFPAT_NP),
                          jnp.asarray(_LT_NP), lin_table.reshape(-1))

    kfn_b = pl.kernel(
        _body_b,
        out_type=jax.ShapeDtypeStruct((_B,), jnp.float32),
        mesh=mesh,
        compiler_params=pltpu.CompilerParams(
            needs_layout_passes=False,
        ),
        scratch_types=[
            pltpu.VMEM((_RPW,), jnp.int32),
            pltpu.VMEM((2 * _SROWS, _D), jnp.float32),
            pltpu.VMEM((_BPW,), jnp.float32),
            pltpu.VMEM((256,), jnp.float32),
            pltpu.VMEM((_BPW,), jnp.float32),
            pltpu.VMEM((16,), jnp.float32),
            pltpu.SemaphoreType.DMA,
            pltpu.SemaphoreType.DMA((2,)),
        ],
    )
    return kfn_b(emb_table, idx_all, lsum, jnp.broadcast_to(bias, (16,)))
